# Initial kernel scaffold; baseline (speedup 1.0000x reference)
import functools

import jax
import jax.numpy as jnp
from jax import lax
from jax.experimental import pallas as pl
from jax.experimental.pallas import tpu as pltpu
from jax.experimental.pallas import tpu_sc as plsc

B = 4
N = 16384
NPOINT = 1024
C = 64
ROWS = B * (N // 128)


def _fps_body(x_ref, y_ref, z_ref, idx_ref, nxyz_ref, dist_ref):
    dist_ref[:, :] = jnp.full((ROWS, 128), 1e10, jnp.float32)
    lane = lax.broadcasted_iota(jnp.int32, (1, 128), 1)
    flat = (lax.broadcasted_iota(jnp.int32, (128, 128), 0) * 128
            + lax.broadcasted_iota(jnp.int32, (128, 128), 1))
    rowb = lax.broadcasted_iota(jnp.int32, (ROWS, 1), 0) // 128

    def body(i, far):
        rowi = jnp.zeros((1, 128), jnp.int32)
        for b in range(B):
            rowi = jnp.where(lane == b, far[b], rowi)
        idx_ref[pl.ds(i, 1), :] = rowi

        cvals = []
        for b in range(B):
            fb = far[b]
            r = fb // 128
            c = fb - r * 128
            msk = lane == c
            sl = pl.ds(b * 128 + r, 1)
            cx = jnp.sum(jnp.where(msk, x_ref[sl, :], 0.0), axis=1, keepdims=True)
            cy = jnp.sum(jnp.where(msk, y_ref[sl, :], 0.0), axis=1, keepdims=True)
            cz = jnp.sum(jnp.where(msk, z_ref[sl, :], 0.0), axis=1, keepdims=True)
            cvals.append((cx, cy, cz))

        rowx = jnp.zeros((1, 128), jnp.float32)
        for b in range(B):
            for d in range(3):
                rowx = jnp.where(lane == 3 * b + d, cvals[b][d], rowx)
        nxyz_ref[pl.ds(i, 1), :] = rowx

        def bsel(vals):
            out = jnp.broadcast_to(vals[B - 1], (ROWS, 1))
            for b in range(B - 2, -1, -1):
                out = jnp.where(rowb == b, vals[b], out)
            return out

        cxv = bsel([cv[0] for cv in cvals])
        cyv = bsel([cv[1] for cv in cvals])
        czv = bsel([cv[2] for cv in cvals])

        dx = x_ref[:, :] - cxv
        dy = y_ref[:, :] - cyv
        dz = z_ref[:, :] - czv
        d = dx * dx + dy * dy + dz * dz
        dm = jnp.minimum(dist_ref[:, :], d)
        dist_ref[:, :] = dm

        nxt = []
        for b in range(B):
            blk = dm[b * 128:(b + 1) * 128, :]
            m = jnp.max(blk)
            cand = jnp.where(blk == m, flat, jnp.int32(1 << 30))
            nxt.append(jnp.min(cand))
        return tuple(nxt)

    z0 = jnp.int32(0)
    lax.fori_loop(0, NPOINT, body, (z0,) * B, unroll=False)


def _fps_call(x, y, z, interpret=False):
    return pl.pallas_call(
        _fps_body,
        out_shape=[
            jax.ShapeDtypeStruct((NPOINT, 128), jnp.int32),
            jax.ShapeDtypeStruct((NPOINT, 128), jnp.float32),
        ],
        scratch_shapes=[pltpu.VMEM((ROWS, 128), jnp.float32)],
        interpret=interpret,
    )(x, y, z)


_SC_MESH = plsc.VectorSubcoreMesh(core_axis_name="c", subcore_axis_name="s")
_NW = 32
_RPW = (B * C) // _NW


@functools.partial(
    pl.kernel,
    out_type=jax.ShapeDtypeStruct((B * C, 8, 128), jnp.float32),
    mesh=_SC_MESH,
    scratch_types=[
        pltpu.VMEM((8, 128), jnp.int32),
        pltpu.VMEM((8, 128), jnp.int32),
        pltpu.VMEM((8, 128), jnp.float32),
        pltpu.SemaphoreType.DMA,
    ],
)
def _sc_gather(feat_hbm, idx_hbm, out_hbm, idxv, posv, rowv, sem):
    cid = lax.axis_index("c")
    sid = lax.axis_index("s")
    wid = sid * 2 + cid
    b = wid // (C // _RPW)
    pltpu.sync_copy(idx_hbm.at[b], idxv)
    row0 = wid * _RPW

    def row_body(j, carry):
        r = row0 + j
        off = r * N
        for q in range(8):
            for t in range(8):
                sl = pl.ds(t * 16, 16)
                posv[q, sl] = idxv[q, sl] + off
        cps = [pltpu.async_copy(feat_hbm.at[posv.at[q]], rowv.at[q], sem)
               for q in range(8)]
        for cp in cps:
            cp.wait()
        pltpu.sync_copy(rowv, out_hbm.at[r])
        return carry

    lax.fori_loop(0, _RPW, row_body, 0, unroll=False)


def kernel(xyz, features):
    x = xyz[:, :, 0].reshape(ROWS, 128)
    y = xyz[:, :, 1].reshape(ROWS, 128)
    z = xyz[:, :, 2].reshape(ROWS, 128)

    idx_raw, nxyz_raw = _fps_call(x, y, z)

    new_xyz = nxyz_raw[:, :3 * B].reshape(NPOINT, B, 3).transpose(1, 0, 2)

    idx3 = idx_raw[:, :B].T.reshape(B, 8, 128)
    feat_flat = features.reshape(-1)
    out = _sc_gather(feat_flat, idx3)
    new_features = out.reshape(B, C, NPOINT)
    return (new_xyz, new_features)


# speedup vs baseline: 5.4495x; 5.4495x over previous
import functools

import jax
import jax.numpy as jnp
from jax import lax
from jax.experimental import pallas as pl
from jax.experimental.pallas import tpu as pltpu
from jax.experimental.pallas import tpu_sc as plsc

B = 4
N = 16384
NPOINT = 1024
C = 64
ROWS = B * (N // 128)


def _fps_body(x_ref, y_ref, z_ref, idx_ref, nxyz_ref, dist_ref):
    dist_ref[:, :] = jnp.full((ROWS, 128), 1e10, jnp.float32)
    lane = lax.broadcasted_iota(jnp.int32, (1, 128), 1)
    flat = (lax.broadcasted_iota(jnp.int32, (128, 128), 0) * 128
            + lax.broadcasted_iota(jnp.int32, (128, 128), 1))
    rowb = lax.broadcasted_iota(jnp.int32, (ROWS, 1), 0) // 128

    def body(i, far):
        rowi = jnp.zeros((1, 128), jnp.int32)
        for b in range(B):
            rowi = jnp.where(lane == b, far[b], rowi)
        idx_ref[pl.ds(i, 1), :] = rowi

        cvals = []
        for b in range(B):
            fb = far[b]
            r = fb // 128
            c = fb - r * 128
            msk = lane == c
            sl = pl.ds(b * 128 + r, 1)
            cx = jnp.sum(jnp.where(msk, x_ref[sl, :], 0.0), axis=1, keepdims=True)
            cy = jnp.sum(jnp.where(msk, y_ref[sl, :], 0.0), axis=1, keepdims=True)
            cz = jnp.sum(jnp.where(msk, z_ref[sl, :], 0.0), axis=1, keepdims=True)
            cvals.append((cx, cy, cz))

        rowx = jnp.zeros((1, 128), jnp.float32)
        for b in range(B):
            for d in range(3):
                rowx = jnp.where(lane == 3 * b + d, cvals[b][d], rowx)
        nxyz_ref[pl.ds(i, 1), :] = rowx

        def bsel(vals):
            out = jnp.broadcast_to(vals[B - 1], (ROWS, 1))
            for b in range(B - 2, -1, -1):
                out = jnp.where(rowb == b, vals[b], out)
            return out

        cxv = bsel([cv[0] for cv in cvals])
        cyv = bsel([cv[1] for cv in cvals])
        czv = bsel([cv[2] for cv in cvals])

        dx = x_ref[:, :] - cxv
        dy = y_ref[:, :] - cyv
        dz = z_ref[:, :] - czv
        d = dx * dx + dy * dy + dz * dz
        dm = jnp.minimum(dist_ref[:, :], d)
        dist_ref[:, :] = dm

        nxt = []
        for b in range(B):
            blk = dm[b * 128:(b + 1) * 128, :]
            m = jnp.max(blk)
            cand = jnp.where(blk == m, flat, jnp.int32(1 << 30))
            nxt.append(jnp.min(cand))
        return tuple(nxt)

    z0 = jnp.int32(0)
    lax.fori_loop(0, NPOINT, body, (z0,) * B, unroll=False)


def _fps_call(x, y, z, interpret=False):
    return pl.pallas_call(
        _fps_body,
        out_shape=[
            jax.ShapeDtypeStruct((NPOINT, 128), jnp.int32),
            jax.ShapeDtypeStruct((NPOINT, 128), jnp.float32),
        ],
        scratch_shapes=[pltpu.VMEM((ROWS, 128), jnp.float32)],
        interpret=interpret,
    )(x, y, z)


_NW = 32
_RPW = (B * C) // _NW


@functools.cache
def _sc_gather_fn():
    mesh = plsc.VectorSubcoreMesh(core_axis_name="c", subcore_axis_name="s")
    return functools.partial(
        pl.kernel,
        out_type=jax.ShapeDtypeStruct((B * C, 8, 128), jnp.float32),
        mesh=mesh,
        scratch_types=[
            pltpu.VMEM((8, 128), jnp.int32),
            pltpu.VMEM((8, 128), jnp.int32),
            pltpu.VMEM((8, 128), jnp.float32),
            pltpu.SemaphoreType.DMA,
        ],
    )(_sc_gather)


def _sc_gather(feat_hbm, idx_hbm, out_hbm, idxv, posv, rowv, sem):
    cid = lax.axis_index("c")
    sid = lax.axis_index("s")
    wid = sid * 2 + cid
    b = wid // (C // _RPW)
    pltpu.sync_copy(idx_hbm.at[b], idxv)
    row0 = wid * _RPW

    def row_body(j, carry):
        r = row0 + j
        off = r * N
        for q in range(8):
            for t in range(8):
                sl = pl.ds(t * 16, 16)
                posv[q, sl] = idxv[q, sl] + off
        cps = [pltpu.async_copy(feat_hbm.at[posv.at[q]], rowv.at[q], sem)
               for q in range(8)]
        for cp in cps:
            cp.wait()
        pltpu.sync_copy(rowv, out_hbm.at[r])
        return carry

    lax.fori_loop(0, _RPW, row_body, 0, unroll=False)


def kernel(xyz, features):
    x = xyz[:, :, 0].reshape(ROWS, 128)
    y = xyz[:, :, 1].reshape(ROWS, 128)
    z = xyz[:, :, 2].reshape(ROWS, 128)

    idx_raw, nxyz_raw = _fps_call(x, y, z)

    new_xyz = nxyz_raw[:, :3 * B].reshape(NPOINT, B, 3).transpose(1, 0, 2)

    idx3 = idx_raw[:, :B].T.reshape(B, 8, 128)
    feat_flat = features.reshape(-1)
    out = _sc_gather_fn()(feat_flat, idx3)
    new_features = out.reshape(B, C, NPOINT)
    return (new_xyz, new_features)


# speedup vs baseline: 11.9705x; 2.1966x over previous
import functools

import jax
import jax.numpy as jnp
from jax import lax
from jax.experimental import pallas as pl
from jax.experimental.pallas import tpu as pltpu
from jax.experimental.pallas import tpu_sc as plsc

B = 4
N = 16384
NPOINT = 1024
C = 64
ROWS = B * (N // 128)


def _red2(op, a):
    return op(op(a, axis=0, keepdims=True), axis=1, keepdims=True)


def _fps_body(x_ref, y_ref, z_ref, idx_ref, nxyz_ref, dist_ref):
    dist_ref[:, :] = jnp.full((ROWS, 128), 1e10, jnp.float32)
    lane = lax.broadcasted_iota(jnp.int32, (1, 128), 1)
    flat = (lax.broadcasted_iota(jnp.int32, (128, 128), 0) * 128
            + lax.broadcasted_iota(jnp.int32, (128, 128), 1))

    def body(i, far):
        rowi = jnp.zeros((1, 128), jnp.int32)
        rowx = jnp.zeros((1, 128), jnp.float32)
        nxt = []
        for b in range(B):
            fb = far[b]
            rowi = jnp.where(lane == b, fb, rowi)
            sl = slice(b * 128, (b + 1) * 128)
            xb = x_ref[sl, :]
            yb = y_ref[sl, :]
            zb = z_ref[sl, :]

            mflat = flat == fb
            cx = _red2(jnp.sum, jnp.where(mflat, xb, 0.0))
            cy = _red2(jnp.sum, jnp.where(mflat, yb, 0.0))
            cz = _red2(jnp.sum, jnp.where(mflat, zb, 0.0))
            for d, cv in enumerate((cx, cy, cz)):
                rowx = jnp.where(lane == 3 * b + d, cv, rowx)

            dx = xb - cx
            dy = yb - cy
            dz = zb - cz
            dsq = dx * dx + dy * dy + dz * dz
            dm = jnp.minimum(dist_ref[sl, :], dsq)
            dist_ref[sl, :] = dm

            m = _red2(jnp.max, dm)
            cand = jnp.where(dm == m, flat, jnp.int32(1 << 30))
            nxt.append(_red2(jnp.min, cand))

        idx_ref[pl.ds(i, 1), :] = rowi
        nxyz_ref[pl.ds(i, 1), :] = rowx
        return tuple(nxt)

    z0 = jnp.zeros((1, 1), jnp.int32)
    lax.fori_loop(0, NPOINT, body, (z0,) * B, unroll=False)


def _fps_call(x, y, z, interpret=False):
    return pl.pallas_call(
        _fps_body,
        out_shape=[
            jax.ShapeDtypeStruct((NPOINT, 128), jnp.int32),
            jax.ShapeDtypeStruct((NPOINT, 128), jnp.float32),
        ],
        scratch_shapes=[pltpu.VMEM((ROWS, 128), jnp.float32)],
        interpret=interpret,
    )(x, y, z)


_NW = 32
_RPW = (B * C) // _NW


@functools.cache
def _sc_gather_fn():
    mesh = plsc.VectorSubcoreMesh(core_axis_name="c", subcore_axis_name="s")
    return functools.partial(
        pl.kernel,
        out_type=jax.ShapeDtypeStruct((B * C, 8, 128), jnp.float32),
        mesh=mesh,
        scratch_types=[
            pltpu.VMEM((8, 128), jnp.int32),
            pltpu.VMEM((8, 128), jnp.int32),
            pltpu.VMEM((8, 128), jnp.float32),
            pltpu.SemaphoreType.DMA,
        ],
    )(_sc_gather)


def _sc_gather(feat_hbm, idx_hbm, out_hbm, idxv, posv, rowv, sem):
    cid = lax.axis_index("c")
    sid = lax.axis_index("s")
    wid = sid * 2 + cid
    b = wid // (C // _RPW)
    pltpu.sync_copy(idx_hbm.at[b], idxv)
    row0 = wid * _RPW

    def row_body(j, carry):
        r = row0 + j
        off = r * N
        for q in range(8):
            for t in range(8):
                sl = pl.ds(t * 16, 16)
                posv[q, sl] = idxv[q, sl] + off
        cps = [pltpu.async_copy(feat_hbm.at[posv.at[q]], rowv.at[q], sem)
               for q in range(8)]
        for cp in cps:
            cp.wait()
        pltpu.sync_copy(rowv, out_hbm.at[r])
        return carry

    lax.fori_loop(0, _RPW, row_body, 0, unroll=False)


def kernel(xyz, features):
    x = xyz[:, :, 0].reshape(ROWS, 128)
    y = xyz[:, :, 1].reshape(ROWS, 128)
    z = xyz[:, :, 2].reshape(ROWS, 128)

    idx_raw, nxyz_raw = _fps_call(x, y, z)

    new_xyz = nxyz_raw[:, :3 * B].reshape(NPOINT, B, 3).transpose(1, 0, 2)

    idx3 = idx_raw[:, :B].T.reshape(B, 8, 128)
    feat_flat = features.reshape(-1)
    out = _sc_gather_fn()(feat_flat, idx3)
    new_features = out.reshape(B, C, NPOINT)
    return (new_xyz, new_features)


# speedup vs baseline: 12.3749x; 1.0338x over previous
import functools

import jax
import jax.numpy as jnp
from jax import lax
from jax.experimental import pallas as pl
from jax.experimental.pallas import tpu as pltpu
from jax.experimental.pallas import tpu_sc as plsc

B = 4
N = 16384
NPOINT = 1024
C = 64
ROWS = B * (N // 128)


def _red2(op, a):
    return op(op(a, axis=0, keepdims=True), axis=1, keepdims=True)


def _fps_body(x_ref, y_ref, z_ref, idx_ref, nxyz_ref, dist_ref):
    dist_ref[:, :] = jnp.full((ROWS, 128), 1e10, jnp.float32)
    lane = lax.broadcasted_iota(jnp.int32, (1, 128), 1)
    flat = (lax.broadcasted_iota(jnp.int32, (128, 128), 0) * 128
            + lax.broadcasted_iota(jnp.int32, (128, 128), 1))

    def body(i, far):
        rowi = jnp.zeros((1, 128), jnp.int32)
        rowx = jnp.zeros((1, 128), jnp.float32)
        nxt = []
        for b in range(B):
            fb = far[b]
            rowi = jnp.where(lane == b, fb, rowi)
            sl = slice(b * 128, (b + 1) * 128)
            xb = x_ref[sl, :]
            yb = y_ref[sl, :]
            zb = z_ref[sl, :]

            mflat = flat == fb
            cx = _red2(jnp.sum, jnp.where(mflat, xb, 0.0))
            cy = _red2(jnp.sum, jnp.where(mflat, yb, 0.0))
            cz = _red2(jnp.sum, jnp.where(mflat, zb, 0.0))
            for d, cv in enumerate((cx, cy, cz)):
                rowx = jnp.where(lane == 3 * b + d, cv, rowx)

            dx = xb - cx
            dy = yb - cy
            dz = zb - cz
            dsq = dx * dx + dy * dy + dz * dz
            dm = jnp.minimum(dist_ref[sl, :], dsq)
            dist_ref[sl, :] = dm

            m = _red2(jnp.max, dm)
            cand = jnp.where(dm == m, flat, jnp.int32(1 << 30))
            nxt.append(_red2(jnp.min, cand))

        idx_ref[pl.ds(i, 1), :] = rowi
        nxyz_ref[pl.ds(i, 1), :] = rowx
        return tuple(nxt)

    z0 = jnp.zeros((1, 1), jnp.int32)
    lax.fori_loop(0, NPOINT, body, (z0,) * B, unroll=2)


def _fps_call(x, y, z, interpret=False):
    return pl.pallas_call(
        _fps_body,
        out_shape=[
            jax.ShapeDtypeStruct((NPOINT, 128), jnp.int32),
            jax.ShapeDtypeStruct((NPOINT, 128), jnp.float32),
        ],
        scratch_shapes=[pltpu.VMEM((ROWS, 128), jnp.float32)],
        interpret=interpret,
    )(x, y, z)


_NW = 32
_RPW = (B * C) // _NW


@functools.cache
def _sc_gather_fn():
    mesh = plsc.VectorSubcoreMesh(core_axis_name="c", subcore_axis_name="s")
    return functools.partial(
        pl.kernel,
        out_type=jax.ShapeDtypeStruct((B * C, 8, 128), jnp.float32),
        mesh=mesh,
        scratch_types=[
            pltpu.VMEM((8, 128), jnp.int32),
            pltpu.VMEM((8, 128), jnp.int32),
            pltpu.VMEM((8, 8, 128), jnp.int32),
            pltpu.VMEM((8, 8, 128), jnp.float32),
            pltpu.SemaphoreType.DMA,
        ],
    )(_sc_gather)


def _sc_gather(feat_hbm, idxr_hbm, out_hbm, posv, idxv, fposv, rowv, sem):
    cid = lax.axis_index("c")
    sid = lax.axis_index("s")
    wid = sid * 2 + cid
    b = wid // (C // _RPW)
    row0 = wid * _RPW

    st = lax.iota(jnp.int32, 16) * 128 + b
    for q in range(8):
        for t in range(8):
            posv[q, pl.ds(t * 16, 16)] = st + (q * 128 + t * 16) * 128
    cps = [pltpu.async_copy(idxr_hbm.at[posv.at[q]], idxv.at[q], sem)
           for q in range(8)]
    for cp in cps:
        cp.wait()

    for j in range(_RPW):
        off = (row0 + j) * N
        for q in range(8):
            for t in range(8):
                sl = pl.ds(t * 16, 16)
                fposv[j, q, sl] = idxv[q, sl] + off
    cps = [pltpu.async_copy(feat_hbm.at[fposv.at[j, q]], rowv.at[j, q], sem)
           for j in range(_RPW) for q in range(8)]
    for cp in cps:
        cp.wait()
    pltpu.sync_copy(rowv, out_hbm.at[pl.ds(row0, _RPW)])


def kernel(xyz, features):
    x = xyz[:, :, 0].reshape(ROWS, 128)
    y = xyz[:, :, 1].reshape(ROWS, 128)
    z = xyz[:, :, 2].reshape(ROWS, 128)

    idx_raw, nxyz_raw = _fps_call(x, y, z)

    new_xyz = nxyz_raw[:, :3 * B].reshape(NPOINT, B, 3).transpose(1, 0, 2)

    feat_flat = features.reshape(-1)
    out = _sc_gather_fn()(feat_flat, idx_raw.reshape(-1))
    new_features = out.reshape(B, C, NPOINT)
    return (new_xyz, new_features)


# speedup vs baseline: 12.6978x; 1.0261x over previous
import functools

import jax
import jax.numpy as jnp
from jax import lax
from jax.experimental import pallas as pl
from jax.experimental.pallas import tpu as pltpu
from jax.experimental.pallas import tpu_sc as plsc

B = 4
N = 16384
NPOINT = 1024
C = 64
ROWS = B * (N // 128)


def _red2(op, a):
    return op(op(a, axis=0, keepdims=True), axis=1, keepdims=True)


def _fps_body(x_ref, y_ref, z_ref, idx_ref, nxyz_ref, dist_ref):
    dist_ref[:, :] = jnp.full((ROWS, 128), 1e10, jnp.float32)
    lane = lax.broadcasted_iota(jnp.int32, (1, 128), 1)
    flat = (lax.broadcasted_iota(jnp.int32, (128, 128), 0) * 128
            + lax.broadcasted_iota(jnp.int32, (128, 128), 1))

    def body(i, far):
        rowi = jnp.zeros((1, 128), jnp.int32)
        rowx = jnp.zeros((1, 128), jnp.float32)
        nxt = []
        for b in range(B):
            fb = far[b]
            rowi = jnp.where(lane == b, fb, rowi)
            sl = slice(b * 128, (b + 1) * 128)
            xb = x_ref[sl, :]
            yb = y_ref[sl, :]
            zb = z_ref[sl, :]

            mflat = flat == fb
            cx = _red2(jnp.sum, jnp.where(mflat, xb, 0.0))
            cy = _red2(jnp.sum, jnp.where(mflat, yb, 0.0))
            cz = _red2(jnp.sum, jnp.where(mflat, zb, 0.0))
            for d, cv in enumerate((cx, cy, cz)):
                rowx = jnp.where(lane == 3 * b + d, cv, rowx)

            dx = xb - cx
            dy = yb - cy
            dz = zb - cz
            dsq = dx * dx + dy * dy + dz * dz
            dm = jnp.minimum(dist_ref[sl, :], dsq)
            dist_ref[sl, :] = dm

            m = _red2(jnp.max, dm)
            cand = jnp.where(dm == m, flat, jnp.int32(1 << 30))
            nxt.append(_red2(jnp.min, cand))

        idx_ref[pl.ds(i, 1), :] = rowi
        nxyz_ref[pl.ds(i, 1), :] = rowx
        return tuple(nxt)

    z0 = jnp.zeros((1, 1), jnp.int32)
    lax.fori_loop(0, NPOINT, body, (z0,) * B, unroll=8)


def _fps_call(x, y, z, interpret=False):
    return pl.pallas_call(
        _fps_body,
        out_shape=[
            jax.ShapeDtypeStruct((NPOINT, 128), jnp.int32),
            jax.ShapeDtypeStruct((NPOINT, 128), jnp.float32),
        ],
        scratch_shapes=[pltpu.VMEM((ROWS, 128), jnp.float32)],
        interpret=interpret,
    )(x, y, z)


_NW = 32
_RPW = (B * C) // _NW


@functools.cache
def _sc_gather_fn():
    mesh = plsc.VectorSubcoreMesh(core_axis_name="c", subcore_axis_name="s")
    return functools.partial(
        pl.kernel,
        out_type=jax.ShapeDtypeStruct((B * C, 8, 128), jnp.float32),
        mesh=mesh,
        scratch_types=[
            pltpu.VMEM((8, 128), jnp.int32),
            pltpu.VMEM((8, 128), jnp.int32),
            pltpu.VMEM((8, 8, 128), jnp.int32),
            pltpu.VMEM((8, 8, 128), jnp.float32),
            pltpu.SemaphoreType.DMA,
        ],
    )(_sc_gather)


def _sc_gather(feat_hbm, idxr_hbm, out_hbm, posv, idxv, fposv, rowv, sem):
    cid = lax.axis_index("c")
    sid = lax.axis_index("s")
    wid = sid * 2 + cid
    b = wid // (C // _RPW)
    row0 = wid * _RPW

    st = lax.iota(jnp.int32, 16) * 128 + b
    for q in range(8):
        for t in range(8):
            posv[q, pl.ds(t * 16, 16)] = st + (q * 128 + t * 16) * 128
    cps = [pltpu.async_copy(idxr_hbm.at[posv.at[q]], idxv.at[q], sem)
           for q in range(8)]
    for cp in cps:
        cp.wait()

    for j in range(_RPW):
        off = (row0 + j) * N
        for q in range(8):
            for t in range(8):
                sl = pl.ds(t * 16, 16)
                fposv[j, q, sl] = idxv[q, sl] + off
    cps = [pltpu.async_copy(feat_hbm.at[fposv.at[j, q]], rowv.at[j, q], sem)
           for j in range(_RPW) for q in range(8)]
    for cp in cps:
        cp.wait()
    pltpu.sync_copy(rowv, out_hbm.at[pl.ds(row0, _RPW)])


def kernel(xyz, features):
    x = xyz[:, :, 0].reshape(ROWS, 128)
    y = xyz[:, :, 1].reshape(ROWS, 128)
    z = xyz[:, :, 2].reshape(ROWS, 128)

    idx_raw, nxyz_raw = _fps_call(x, y, z)

    new_xyz = nxyz_raw[:, :3 * B].reshape(NPOINT, B, 3).transpose(1, 0, 2)

    feat_flat = features.reshape(-1)
    out = _sc_gather_fn()(feat_flat, idx_raw.reshape(-1))
    new_features = out.reshape(B, C, NPOINT)
    return (new_xyz, new_features)


# speedup vs baseline: 15.3458x; 1.2085x over previous
import functools

import jax
import jax.numpy as jnp
from jax import lax
from jax.experimental import pallas as pl
from jax.experimental.pallas import tpu as pltpu
from jax.experimental.pallas import tpu_sc as plsc

B = 4
N = 16384
NPOINT = 1024
C = 64
ROWS = B * (N // 128)


def _red2(op, a):
    return op(op(a, axis=0, keepdims=True), axis=1, keepdims=True)


def _fps_body(x_ref, y_ref, z_ref, idx_ref, nxyz_ref, dist_ref):
    dist_ref[:, :] = jnp.full((ROWS, 128), 1e10, jnp.float32)
    lane = lax.broadcasted_iota(jnp.int32, (1, 128), 1)
    flat = (lax.broadcasted_iota(jnp.int32, (128, 128), 0) * 128
            + lax.broadcasted_iota(jnp.int32, (128, 128), 1)
            ).astype(jnp.float32)

    def body(i, far):
        rowi = jnp.zeros((1, 128), jnp.int32)
        rowx = jnp.zeros((1, 128), jnp.float32)
        nxt = []
        for b in range(B):
            fb = far[b]
            rowi = jnp.where(lane == b, fb.astype(jnp.int32), rowi)
            sl = slice(b * 128, (b + 1) * 128)
            xb = x_ref[sl, :]
            yb = y_ref[sl, :]
            zb = z_ref[sl, :]

            mflat = flat == fb
            cx = _red2(jnp.sum, jnp.where(mflat, xb, 0.0))
            cy = _red2(jnp.sum, jnp.where(mflat, yb, 0.0))
            cz = _red2(jnp.sum, jnp.where(mflat, zb, 0.0))
            for d, cv in enumerate((cx, cy, cz)):
                rowx = jnp.where(lane == 3 * b + d, cv, rowx)

            dx = xb - cx
            dy = yb - cy
            dz = zb - cz
            dsq = dx * dx + dy * dy + dz * dz
            dm = jnp.minimum(dist_ref[sl, :], dsq)
            dist_ref[sl, :] = dm

            m = _red2(jnp.max, dm)
            cand = jnp.where(dm == m, flat, jnp.float32(1 << 30))
            nxt.append(_red2(jnp.min, cand))

        idx_ref[pl.ds(i, 1), :] = rowi
        nxyz_ref[pl.ds(i, 1), :] = rowx
        return tuple(nxt)

    z0 = jnp.zeros((1, 1), jnp.float32)
    lax.fori_loop(0, NPOINT, body, (z0,) * B, unroll=8)


def _fps_call(x, y, z, interpret=False):
    return pl.pallas_call(
        _fps_body,
        out_shape=[
            jax.ShapeDtypeStruct((NPOINT, 128), jnp.int32),
            jax.ShapeDtypeStruct((NPOINT, 128), jnp.float32),
        ],
        scratch_shapes=[pltpu.VMEM((ROWS, 128), jnp.float32)],
        interpret=interpret,
    )(x, y, z)


_NW = 32
_RPW = (B * C) // _NW


@functools.cache
def _sc_gather_fn():
    mesh = plsc.VectorSubcoreMesh(core_axis_name="c", subcore_axis_name="s")
    return functools.partial(
        pl.kernel,
        out_type=jax.ShapeDtypeStruct((B * C, 8, 128), jnp.float32),
        mesh=mesh,
        scratch_types=[
            pltpu.VMEM((8, 128), jnp.int32),
            pltpu.VMEM((8, 128), jnp.int32),
            pltpu.VMEM((8, 8, 128), jnp.int32),
            pltpu.VMEM((8, 8, 128), jnp.float32),
            pltpu.SemaphoreType.DMA,
        ],
    )(_sc_gather)


def _sc_gather(feat_hbm, idxr_hbm, out_hbm, posv, idxv, fposv, rowv, sem):
    cid = lax.axis_index("c")
    sid = lax.axis_index("s")
    wid = sid * 2 + cid
    b = wid // (C // _RPW)
    row0 = wid * _RPW

    st = lax.iota(jnp.int32, 16) * 128 + b
    for q in range(8):
        for t in range(8):
            posv[q, pl.ds(t * 16, 16)] = st + (q * 128 + t * 16) * 128
    cps = [pltpu.async_copy(idxr_hbm.at[posv.at[q]], idxv.at[q], sem)
           for q in range(8)]
    for cp in cps:
        cp.wait()

    for j in range(_RPW):
        off = (row0 + j) * N
        for q in range(8):
            for t in range(8):
                sl = pl.ds(t * 16, 16)
                fposv[j, q, sl] = idxv[q, sl] + off
    cps = [pltpu.async_copy(feat_hbm.at[fposv.at[j, q]], rowv.at[j, q], sem)
           for j in range(_RPW) for q in range(8)]
    for cp in cps:
        cp.wait()
    pltpu.sync_copy(rowv, out_hbm.at[pl.ds(row0, _RPW)])


def kernel(xyz, features):
    x = xyz[:, :, 0].reshape(ROWS, 128)
    y = xyz[:, :, 1].reshape(ROWS, 128)
    z = xyz[:, :, 2].reshape(ROWS, 128)

    idx_raw, nxyz_raw = _fps_call(x, y, z)

    new_xyz = nxyz_raw[:, :3 * B].reshape(NPOINT, B, 3).transpose(1, 0, 2)

    feat_flat = features.reshape(-1)
    out = _sc_gather_fn()(feat_flat, idx_raw.reshape(-1))
    new_features = out.reshape(B, C, NPOINT)
    return (new_xyz, new_features)


# speedup vs baseline: 15.9127x; 1.0369x over previous
import functools

import jax
import jax.numpy as jnp
from jax import lax
from jax.experimental import pallas as pl
from jax.experimental.pallas import tpu as pltpu
from jax.experimental.pallas import tpu_sc as plsc

B = 4
N = 16384
NPOINT = 1024
C = 64
ROWS = B * (N // 128)


def _red2(op, a):
    return op(op(a, axis=0, keepdims=True), axis=1, keepdims=True)


def _fps_body(x_ref, y_ref, z_ref, idx_ref, nxyz_ref, dist_ref):
    dist_ref[:, :] = jnp.full((ROWS, 128), 1e10, jnp.float32)
    lane = lax.broadcasted_iota(jnp.int32, (1, 128), 1)
    flat = (lax.broadcasted_iota(jnp.int32, (128, 128), 0) * 128
            + lax.broadcasted_iota(jnp.int32, (128, 128), 1)
            ).astype(jnp.float32)

    def body(i, far):
        rowi = jnp.zeros((1, 128), jnp.int32)
        rowx = jnp.zeros((1, 128), jnp.float32)
        nxt = []
        for b in range(B):
            fb = far[b]
            rowi = jnp.where(lane == b, fb.astype(jnp.int32), rowi)
            sl = slice(b * 128, (b + 1) * 128)
            xb = x_ref[sl, :]
            yb = y_ref[sl, :]
            zb = z_ref[sl, :]

            mflat = flat == fb
            cx = _red2(jnp.sum, jnp.where(mflat, xb, 0.0))
            cy = _red2(jnp.sum, jnp.where(mflat, yb, 0.0))
            cz = _red2(jnp.sum, jnp.where(mflat, zb, 0.0))
            for d, cv in enumerate((cx, cy, cz)):
                rowx = jnp.where(lane == 3 * b + d, cv, rowx)

            dx = xb - cx
            dy = yb - cy
            dz = zb - cz
            dsq = dx * dx + dy * dy + dz * dz
            dm = jnp.minimum(dist_ref[sl, :], dsq)
            dist_ref[sl, :] = dm

            colmax = jnp.max(dm, axis=0, keepdims=True)
            m = jnp.max(colmax, axis=1, keepdims=True)
            rowcand = jnp.min(jnp.where(dm == colmax, flat,
                                        jnp.float32(1 << 30)),
                              axis=0, keepdims=True)
            lcand = jnp.where(colmax == m, rowcand, jnp.float32(1 << 30))
            nxt.append(jnp.min(lcand, axis=1, keepdims=True))

        idx_ref[pl.ds(i, 1), :] = rowi
        nxyz_ref[pl.ds(i, 1), :] = rowx
        return tuple(nxt)

    z0 = jnp.zeros((1, 1), jnp.float32)
    lax.fori_loop(0, NPOINT, body, (z0,) * B, unroll=8)


def _fps_call(x, y, z, interpret=False):
    return pl.pallas_call(
        _fps_body,
        out_shape=[
            jax.ShapeDtypeStruct((NPOINT, 128), jnp.int32),
            jax.ShapeDtypeStruct((NPOINT, 128), jnp.float32),
        ],
        scratch_shapes=[pltpu.VMEM((ROWS, 128), jnp.float32)],
        interpret=interpret,
    )(x, y, z)


_NW = 32
_RPW = (B * C) // _NW


@functools.cache
def _sc_gather_fn():
    mesh = plsc.VectorSubcoreMesh(core_axis_name="c", subcore_axis_name="s")
    return functools.partial(
        pl.kernel,
        out_type=jax.ShapeDtypeStruct((B * C, 8, 128), jnp.float32),
        mesh=mesh,
        scratch_types=[
            pltpu.VMEM((8, 128), jnp.int32),
            pltpu.VMEM((8, 128), jnp.int32),
            pltpu.VMEM((8, 8, 128), jnp.int32),
            pltpu.VMEM((8, 8, 128), jnp.float32),
            pltpu.SemaphoreType.DMA,
        ],
    )(_sc_gather)


def _sc_gather(feat_hbm, idxr_hbm, out_hbm, posv, idxv, fposv, rowv, sem):
    cid = lax.axis_index("c")
    sid = lax.axis_index("s")
    wid = sid * 2 + cid
    b = wid // (C // _RPW)
    row0 = wid * _RPW

    st = lax.iota(jnp.int32, 16) * 128 + b
    for q in range(8):
        for t in range(8):
            posv[q, pl.ds(t * 16, 16)] = st + (q * 128 + t * 16) * 128
    cps = [pltpu.async_copy(idxr_hbm.at[posv.at[q]], idxv.at[q], sem)
           for q in range(8)]
    for cp in cps:
        cp.wait()

    for j in range(_RPW):
        off = (row0 + j) * N
        for q in range(8):
            for t in range(8):
                sl = pl.ds(t * 16, 16)
                fposv[j, q, sl] = idxv[q, sl] + off
    cps = [pltpu.async_copy(feat_hbm.at[fposv.at[j, q]], rowv.at[j, q], sem)
           for j in range(_RPW) for q in range(8)]
    for cp in cps:
        cp.wait()
    pltpu.sync_copy(rowv, out_hbm.at[pl.ds(row0, _RPW)])


def kernel(xyz, features):
    x = xyz[:, :, 0].reshape(ROWS, 128)
    y = xyz[:, :, 1].reshape(ROWS, 128)
    z = xyz[:, :, 2].reshape(ROWS, 128)

    idx_raw, nxyz_raw = _fps_call(x, y, z)

    new_xyz = nxyz_raw[:, :3 * B].reshape(NPOINT, B, 3).transpose(1, 0, 2)

    feat_flat = features.reshape(-1)
    out = _sc_gather_fn()(feat_flat, idx_raw.reshape(-1))
    new_features = out.reshape(B, C, NPOINT)
    return (new_xyz, new_features)


# speedup vs baseline: 16.6023x; 1.0433x over previous
import functools

import jax
import jax.numpy as jnp
from jax import lax
from jax.experimental import pallas as pl
from jax.experimental.pallas import tpu as pltpu
from jax.experimental.pallas import tpu_sc as plsc

B = 4
N = 16384
NPOINT = 1024
C = 64
ROWS = B * (N // 128)


def _red2(op, a):
    return op(op(a, axis=0, keepdims=True), axis=1, keepdims=True)


def _fps_body(x_ref, y_ref, z_ref, idx_ref, nxyz_ref, dist_ref):
    dist_ref[:, :] = jnp.full((ROWS, 128), 1e10, jnp.float32)
    lane = lax.broadcasted_iota(jnp.int32, (1, 128), 1)
    flat = (lax.broadcasted_iota(jnp.int32, (128, 128), 0) * 128
            + lax.broadcasted_iota(jnp.int32, (128, 128), 1)
            ).astype(jnp.float32)

    def body(i, st):
        rowi = jnp.zeros((1, 128), jnp.int32)
        rowx = jnp.zeros((1, 128), jnp.float32)
        nxt = []
        for b in range(B):
            fb, lc, colx, coly, colz = st[b]
            rowi = jnp.where(lane == b, fb.astype(jnp.int32), rowi)

            lm = lc == fb
            cx = jnp.sum(jnp.where(lm, colx, 0.0), axis=1, keepdims=True)
            cy = jnp.sum(jnp.where(lm, coly, 0.0), axis=1, keepdims=True)
            cz = jnp.sum(jnp.where(lm, colz, 0.0), axis=1, keepdims=True)
            for d, cv in enumerate((cx, cy, cz)):
                rowx = jnp.where(lane == 3 * b + d, cv, rowx)

            sl = slice(b * 128, (b + 1) * 128)
            xb = x_ref[sl, :]
            yb = y_ref[sl, :]
            zb = z_ref[sl, :]
            dx = xb - cx
            dy = yb - cy
            dz = zb - cz
            dsq = dx * dx + dy * dy + dz * dz
            dm = jnp.minimum(dist_ref[sl, :], dsq)
            dist_ref[sl, :] = dm

            colmax = jnp.max(dm, axis=0, keepdims=True)
            m = jnp.max(colmax, axis=1, keepdims=True)
            rowcand = jnp.min(jnp.where(dm == colmax, flat,
                                        jnp.float32(1 << 30)),
                              axis=0, keepdims=True)
            mr = flat == rowcand
            ncolx = jnp.sum(jnp.where(mr, xb, 0.0), axis=0, keepdims=True)
            ncoly = jnp.sum(jnp.where(mr, yb, 0.0), axis=0, keepdims=True)
            ncolz = jnp.sum(jnp.where(mr, zb, 0.0), axis=0, keepdims=True)
            nlc = jnp.where(colmax == m, rowcand, jnp.float32(1 << 30))
            nfb = jnp.min(nlc, axis=1, keepdims=True)
            nxt.append((nfb, nlc, ncolx, ncoly, ncolz))

        idx_ref[pl.ds(i, 1), :] = rowi
        nxyz_ref[pl.ds(i, 1), :] = rowx
        return tuple(nxt)

    f0 = jnp.zeros((1, 1), jnp.float32)
    lc0 = jnp.where(lane == 0, 0.0, jnp.float32(1 << 30))
    init = tuple(
        (f0, lc0, x_ref[b * 128:b * 128 + 1, :],
         y_ref[b * 128:b * 128 + 1, :], z_ref[b * 128:b * 128 + 1, :])
        for b in range(B))
    lax.fori_loop(0, NPOINT, body, init, unroll=8)


def _fps_call(x, y, z, interpret=False):
    return pl.pallas_call(
        _fps_body,
        out_shape=[
            jax.ShapeDtypeStruct((NPOINT, 128), jnp.int32),
            jax.ShapeDtypeStruct((NPOINT, 128), jnp.float32),
        ],
        scratch_shapes=[pltpu.VMEM((ROWS, 128), jnp.float32)],
        interpret=interpret,
    )(x, y, z)


_NW = 32
_RPW = (B * C) // _NW


@functools.cache
def _sc_gather_fn():
    mesh = plsc.VectorSubcoreMesh(core_axis_name="c", subcore_axis_name="s")
    return functools.partial(
        pl.kernel,
        out_type=jax.ShapeDtypeStruct((B * C, 8, 128), jnp.float32),
        mesh=mesh,
        scratch_types=[
            pltpu.VMEM((8, 128), jnp.int32),
            pltpu.VMEM((8, 128), jnp.int32),
            pltpu.VMEM((8, 8, 128), jnp.int32),
            pltpu.VMEM((8, 8, 128), jnp.float32),
            pltpu.SemaphoreType.DMA,
        ],
    )(_sc_gather)


def _sc_gather(feat_hbm, idxr_hbm, out_hbm, posv, idxv, fposv, rowv, sem):
    cid = lax.axis_index("c")
    sid = lax.axis_index("s")
    wid = sid * 2 + cid
    b = wid // (C // _RPW)
    row0 = wid * _RPW

    st = lax.iota(jnp.int32, 16) * 128 + b
    for q in range(8):
        for t in range(8):
            posv[q, pl.ds(t * 16, 16)] = st + (q * 128 + t * 16) * 128
    cps = [pltpu.async_copy(idxr_hbm.at[posv.at[q]], idxv.at[q], sem)
           for q in range(8)]
    for cp in cps:
        cp.wait()

    for j in range(_RPW):
        off = (row0 + j) * N
        for q in range(8):
            for t in range(8):
                sl = pl.ds(t * 16, 16)
                fposv[j, q, sl] = idxv[q, sl] + off
    cps = [pltpu.async_copy(feat_hbm.at[fposv.at[j, q]], rowv.at[j, q], sem)
           for j in range(_RPW) for q in range(8)]
    for cp in cps:
        cp.wait()
    pltpu.sync_copy(rowv, out_hbm.at[pl.ds(row0, _RPW)])


def kernel(xyz, features):
    x = xyz[:, :, 0].reshape(ROWS, 128)
    y = xyz[:, :, 1].reshape(ROWS, 128)
    z = xyz[:, :, 2].reshape(ROWS, 128)

    idx_raw, nxyz_raw = _fps_call(x, y, z)

    new_xyz = nxyz_raw[:, :3 * B].reshape(NPOINT, B, 3).transpose(1, 0, 2)

    feat_flat = features.reshape(-1)
    out = _sc_gather_fn()(feat_flat, idx_raw.reshape(-1))
    new_features = out.reshape(B, C, NPOINT)
    return (new_xyz, new_features)
